# baseline (speedup 1.0000x reference)
import jax
import jax.numpy as jnp
from jax import lax
from jax.experimental import pallas as pl
from jax.experimental.pallas import tpu as pltpu
from jax.experimental.pallas import tpu_sc as plsc

N = 100000
E = 1600000
BR = 2000

TILE_E = 50176
EPAD = 32 * TILE_E
NV = 50000
CH = 256
NF = 391
NFP = 400
NDEGT = 100096
LLEN = E + 3200
NACC = NF * CH
D1 = 256
D2 = 128

_MESH = plsc.VectorSubcoreMesh(core_axis_name="c", subcore_axis_name="s")
_CP = pltpu.CompilerParams(needs_layout_passes=False)


def _count_body(src_hbm, dst_hbm, zcf_hbm, zdeg_hbm, counts_hbm, degp_hbm,
                s_v, d_v, cf_v, deg_v):
    c = lax.axis_index("c")
    s = lax.axis_index("s")
    w = c * 16 + s
    lane = lax.iota(jnp.int32, 16)

    pltpu.sync_copy(zcf_hbm, cf_v)
    pltpu.sync_copy(zdeg_hbm, deg_v)
    base = w * TILE_E

    def chunk_body(j, carry):
        eoff = pl.multiple_of(base + j * 512, 8)
        pltpu.sync_copy(src_hbm.at[pl.ds(eoff, 512)], s_v)
        pltpu.sync_copy(dst_hbm.at[pl.ds(eoff, 512)], d_v)

        def row_body(v, carry2):
            off = pl.multiple_of(v * 16, 8)
            vd = d_v[pl.ds(off, 16)]
            valid = (j * 512 + off + lane) < NV
            f = lax.shift_right_logical(vd, 8)
            occ, last = plsc.scan_count(f, valid)
            cnt = plsc.load_gather(cf_v, [f])
            plsc.store_scatter(cf_v, [f], cnt + occ, mask=last)
            docc, dlast = plsc.scan_count(vd, valid)
            dcnt = plsc.load_gather(deg_v, [vd])
            plsc.store_scatter(deg_v, [vd], dcnt + docc, mask=dlast)
            return carry2

        return lax.fori_loop(0, 32, row_body, carry)

    lax.fori_loop(0, 98, chunk_body, 0)

    pltpu.sync_copy(cf_v, counts_hbm.at[pl.ds(pl.multiple_of(w * NFP, 8), NFP)])
    pltpu.sync_copy(deg_v, degp_hbm.at[w])


_count_call = pl.kernel(
    _count_body,
    out_type=[
        jax.ShapeDtypeStruct((32 * NFP,), jnp.int32),
        jax.ShapeDtypeStruct((32, NDEGT), jnp.int32),
    ],
    mesh=_MESH,
    scratch_types=[
        pltpu.VMEM((512,), jnp.int32),
        pltpu.VMEM((512,), jnp.int32),
        pltpu.VMEM((NFP,), jnp.int32),
        pltpu.VMEM((NDEGT,), jnp.int32),
    ],
    compiler_params=_CP,
)


def _plan_body(counts_ref, bases_ref, meta_ref):
    cnts = counts_ref[...].astype(jnp.float32)
    totals = jnp.sum(cnts, axis=0, keepdims=True)
    alloc = jnp.floor((totals + 7.0) * 0.125) * 8.0
    ri = lax.broadcasted_iota(jnp.int32, (NFP, NFP), 0)
    ci = lax.broadcasted_iota(jnp.int32, (NFP, NFP), 1)
    m_excl = (ri < ci).astype(jnp.float32)
    segstart = jnp.dot(alloc, m_excl,
                       preferred_element_type=jnp.float32)
    ri32 = lax.broadcasted_iota(jnp.int32, (32, 32), 0)
    ci32 = lax.broadcasted_iota(jnp.int32, (32, 32), 1)
    a_excl = (ci32 < ri32).astype(jnp.float32)
    colcum = jnp.dot(a_excl, cnts,
                     preferred_element_type=jnp.float32)
    bases_ref[...] = (segstart + colcum).astype(jnp.int32)
    rio = lax.broadcasted_iota(jnp.int32, (8, NFP), 0)
    seg_b = jnp.broadcast_to(segstart, (8, NFP))
    tot_b = jnp.broadcast_to(totals, (8, NFP))
    meta_ref[...] = jnp.where(
        rio == 0, seg_b, jnp.where(rio == 1, tot_b, 0.0)).astype(jnp.int32)


def _plan(counts2d):
    return pl.pallas_call(
        _plan_body,
        out_shape=(
            jax.ShapeDtypeStruct((32, NFP), jnp.int32),
            jax.ShapeDtypeStruct((8, NFP), jnp.int32),
        ),
    )(counts2d)


def _place_body(src_hbm, dst_hbm, zcf_hbm, bases_hbm, lists_hbm,
                s_v, d_v, cf_v, b_v, pkst, post, sem):
    c = lax.axis_index("c")
    s = lax.axis_index("s")
    w = c * 16 + s
    lane = lax.iota(jnp.int32, 16)

    pltpu.sync_copy(zcf_hbm, cf_v)
    pltpu.sync_copy(bases_hbm.at[pl.ds(pl.multiple_of(w * NFP, 8), NFP)], b_v)
    base = w * TILE_E

    def chunk_body(j, carry):
        eoff = pl.multiple_of(base + j * 512, 8)
        pltpu.sync_copy(src_hbm.at[pl.ds(eoff, 512)], s_v)
        pltpu.sync_copy(dst_hbm.at[pl.ds(eoff, 512)], d_v)

        def row_body(v, carry2):
            off = pl.multiple_of(v * 16, 8)
            vd = d_v[pl.ds(off, 16)]
            vs = s_v[pl.ds(off, 16)]
            valid = (j * 512 + off + lane) < NV
            f = lax.shift_right_logical(vd, 8)
            occ, last = plsc.scan_count(f, valid)
            cnt = plsc.load_gather(cf_v, [f])
            plsc.store_scatter(cf_v, [f], cnt + occ, mask=last)
            gb = plsc.load_gather(b_v, [f])
            pos = jnp.where(valid, gb + cnt + occ - 1, -1)
            pk = jnp.bitwise_or(vs, jnp.left_shift(jnp.bitwise_and(vd, 255), 17))
            pkst[pl.ds(off, 16)] = pk
            post[pl.ds(off, 16)] = pos
            return carry2

        lax.fori_loop(0, 32, row_body, carry)
        pltpu.async_copy(
            pkst, lists_hbm.at[plsc.Indices(post, ignored_value=-1)], sem
        ).wait()
        return carry

    lax.fori_loop(0, 98, chunk_body, 0)


_place_call = pl.kernel(
    _place_body,
    out_type=jax.ShapeDtypeStruct((LLEN,), jnp.int32),
    mesh=_MESH,
    scratch_types=[
        pltpu.VMEM((512,), jnp.int32),
        pltpu.VMEM((512,), jnp.int32),
        pltpu.VMEM((NFP,), jnp.int32),
        pltpu.VMEM((NFP,), jnp.int32),
        pltpu.VMEM((512,), jnp.int32),
        pltpu.VMEM((512,), jnp.int32),
        pltpu.SemaphoreType.DMA,
    ],
    compiler_params=_CP,
)


def _make_scatter(D):
    def body(g_hbm, lists_hbm, meta_hbm, zacc_hbm, out_hbm,
             pk_v, gidx_v, mt_v, rows_v, acc_v, sem):
        c = lax.axis_index("c")
        s = lax.axis_index("s")
        w = c * 16 + s
        lane = lax.iota(jnp.int32, 16)
        rowidx = [lane + v8 * 16 for v8 in range(8)]

        pltpu.sync_copy(meta_hbm, mt_v)
        for r in range(13):
            fidx = r * 32 + w

            @pl.when(fidx <= NF - 1)
            def _chunk():
                pltpu.sync_copy(zacc_hbm, acc_v.at[pl.ds(0, 128)])
                pltpu.sync_copy(zacc_hbm, acc_v.at[pl.ds(128, 128)])
                fvec = jnp.zeros((16,), jnp.int32) + fidx
                sb_v = plsc.load_gather(mt_v, [fvec])
                ln_v = plsc.load_gather(mt_v, [fvec + NFP])
                sbase = jnp.sum(jnp.where(lane == 0, sb_v, 0))
                n = jnp.sum(jnp.where(lane == 0, ln_v, 0))
                nch = lax.div(n + 127, 128)

                def ch_body(j, carry):
                    joff = pl.multiple_of(sbase + j * 128, 8)
                    pltpu.sync_copy(lists_hbm.at[pl.ds(joff, 128)], pk_v)
                    rem = n - j * 128
                    dvs = []
                    for v8 in range(8):
                        pkv = pk_v[pl.ds(v8 * 16, 16)]
                        m = (lane + v8 * 16) < rem
                        sv = jnp.where(m, jnp.bitwise_and(pkv, 131071), 0)
                        dv = jnp.where(m, lax.shift_right_logical(pkv, 17), CH)
                        gidx_v[pl.ds(v8 * 16, 16)] = sv
                        dvs.append(dv)
                    pltpu.async_copy(g_hbm.at[gidx_v], rows_v, sem).wait()

                    def col_body(cc, carry2):
                        cful = jnp.zeros((16,), jnp.int32) + cc
                        for v8 in range(8):
                            vals = plsc.load_gather(rows_v, [rowidx[v8], cful])
                            plsc.addupdate_scatter(
                                acc_v, [carry2[v8], cful], vals)
                        return carry2

                    lax.fori_loop(0, D, col_body, tuple(dvs))
                    return carry

                lax.fori_loop(0, nch, ch_body, 0)
                ooff = pl.multiple_of(fidx * CH, 8)
                pltpu.sync_copy(acc_v.at[pl.ds(0, 128)],
                                out_hbm.at[pl.ds(ooff, 128)])
                pltpu.sync_copy(acc_v.at[pl.ds(128, 128)],
                                out_hbm.at[pl.ds(ooff + 128, 128)])

    return pl.kernel(
        body,
        out_type=jax.ShapeDtypeStruct((NACC, D), jnp.float32),
        mesh=_MESH,
        scratch_types=[
            pltpu.VMEM((128,), jnp.int32),
            pltpu.VMEM((128,), jnp.int32),
            pltpu.VMEM((2 * NFP,), jnp.int32),
            pltpu.VMEM((128, D), jnp.float32),
            pltpu.VMEM((CH + 8, D), jnp.float32),
            pltpu.SemaphoreType.DMA,
        ],
        compiler_params=_CP,
    )


_scatter_d1 = _make_scatter(D1)
_scatter_d2 = _make_scatter(D2)


def _deg_body(degp_ref, o_ref):
    total = jnp.sum(degp_ref[...], axis=0, keepdims=True)
    o_ref[...] = lax.rsqrt(total.astype(jnp.float32) + 1.0)


def _dinv(degp):
    bw = 2944
    return pl.pallas_call(
        _deg_body,
        grid=(NDEGT // bw,),
        in_specs=[pl.BlockSpec((32, bw), lambda i: (0, i))],
        out_specs=pl.BlockSpec((1, bw), lambda i: (0, i)),
        out_shape=jax.ShapeDtypeStruct((1, NDEGT), jnp.float32),
    )(degp)


def _mm_scale_body(x_ref, w_ref, dinv_ref, o_ref):
    o_ref[...] = (
        jnp.dot(x_ref[...], w_ref[...], preferred_element_type=jnp.float32)
        * dinv_ref[...]
    )


def _mm_scale(x, w, dinv):
    n, din = x.shape
    dout = w.shape[1]
    return pl.pallas_call(
        _mm_scale_body,
        grid=(n // BR,),
        in_specs=[
            pl.BlockSpec((BR, din), lambda i: (i, 0)),
            pl.BlockSpec((din, dout), lambda i: (0, 0)),
            pl.BlockSpec((BR, 1), lambda i: (i, 0)),
        ],
        out_specs=pl.BlockSpec((BR, dout), lambda i: (i, 0)),
        out_shape=jax.ShapeDtypeStruct((n, dout), jnp.float32),
    )(x, w, dinv)


def _post_mm_scale_body(acc_ref, g_ref, dinv_ref, b_ref, w_ref, o_ref):
    h = jnp.maximum((acc_ref[...] + g_ref[...]) * dinv_ref[...] + b_ref[...], 0.0)
    o_ref[...] = (
        jnp.dot(h, w_ref[...], preferred_element_type=jnp.float32) * dinv_ref[...]
    )


def _post_mm_scale(acc, g, dinv, b, w):
    n, din = g.shape
    dout = w.shape[1]
    return pl.pallas_call(
        _post_mm_scale_body,
        grid=(n // BR,),
        in_specs=[
            pl.BlockSpec((BR, din), lambda i: (i, 0)),
            pl.BlockSpec((BR, din), lambda i: (i, 0)),
            pl.BlockSpec((BR, 1), lambda i: (i, 0)),
            pl.BlockSpec((1, din), lambda i: (0, 0)),
            pl.BlockSpec((din, dout), lambda i: (0, 0)),
        ],
        out_specs=pl.BlockSpec((BR, dout), lambda i: (i, 0)),
        out_shape=jax.ShapeDtypeStruct((n, dout), jnp.float32),
    )(acc, g, dinv, b.reshape(1, din), w)


def _final_body(acc_ref, g_ref, dinv_ref, b_ref, o_ref):
    i = pl.program_id(0)
    h = jnp.maximum((acc_ref[...] + g_ref[...]) * dinv_ref[...] + b_ref[...], 0.0)
    part = jnp.sum(h, axis=0, keepdims=True) * (1.0 / N)

    @pl.when(i == 0)
    def _init():
        o_ref[...] = jnp.zeros_like(o_ref)

    o_ref[...] += part


def _final_mean(acc, g, dinv, b):
    n, d = g.shape
    return pl.pallas_call(
        _final_body,
        grid=(n // BR,),
        in_specs=[
            pl.BlockSpec((BR, d), lambda i: (i, 0)),
            pl.BlockSpec((BR, d), lambda i: (i, 0)),
            pl.BlockSpec((BR, 1), lambda i: (i, 0)),
            pl.BlockSpec((1, d), lambda i: (0, 0)),
        ],
        out_specs=pl.BlockSpec((1, d), lambda i: (0, 0)),
        out_shape=jax.ShapeDtypeStruct((1, d), jnp.float32),
    )(acc, g, dinv, b.reshape(1, d))


@jax.jit
def kernel(x, edge_index, W1, b1, W2, b2):
    w1p = jnp.pad(W1, ((0, 0), (0, D1 - W1.shape[1])))
    b1p = jnp.pad(b1, (0, D1 - b1.shape[0]))
    w2p = jnp.pad(W2, ((0, D1 - W2.shape[0]), (0, D2 - W2.shape[1])))
    b2p = jnp.pad(b2, (0, D2 - b2.shape[0]))
    srcp = jnp.pad(edge_index[0], (0, EPAD - E))
    dstp = jnp.pad(edge_index[1], (0, EPAD - E))
    zcf = jnp.zeros((NFP,), jnp.int32)
    zdeg = jnp.zeros((NDEGT,), jnp.int32)

    counts, degp = _count_call(srcp, dstp, zcf, zdeg)
    bases, meta = _plan(counts.reshape(32, NFP))
    lists = _place_call(srcp, dstp, zcf, bases.reshape(-1))

    dinv = _dinv(degp)[:, :N].reshape(N, 1)
    g1 = _mm_scale(x, w1p, dinv)
    acc1 = _scatter_d1(g1, lists, meta.reshape(-1)[:2 * NFP],
                       jnp.zeros((128, D1), jnp.float32))
    g2 = _post_mm_scale(acc1[:N], g1, dinv, b1p, w2p)
    acc2 = _scatter_d2(g2, lists, meta.reshape(-1)[:2 * NFP],
                       jnp.zeros((128, D2), jnp.float32))
    return _final_mean(acc2[:N], g2, dinv, b2p)[:, :100]


# speedup vs baseline: 1.4247x; 1.4247x over previous
import jax
import jax.numpy as jnp
from jax import lax
from jax.experimental import pallas as pl
from jax.experimental.pallas import tpu as pltpu
from jax.experimental.pallas import tpu_sc as plsc

N = 100000
E = 1600000
BR = 2000

TILE_E = 50176
EPAD = 32 * TILE_E
NV = 50000
CH = 256
NF = 391
NFP = 400
NDEGT = 100096
LLEN = E + 3200
NACC = NF * CH
D1 = 256
D2 = 128
DU = 128

_MESH = plsc.VectorSubcoreMesh(core_axis_name="c", subcore_axis_name="s")
_CP = pltpu.CompilerParams(needs_layout_passes=False)


def _count_body(src_hbm, dst_hbm, zcf_hbm, zdeg_hbm, counts_hbm, degp_hbm,
                s_v, d_v, cf_v, deg_v):
    c = lax.axis_index("c")
    s = lax.axis_index("s")
    w = c * 16 + s
    lane = lax.iota(jnp.int32, 16)

    pltpu.sync_copy(zcf_hbm, cf_v)
    pltpu.sync_copy(zdeg_hbm, deg_v)
    base = w * TILE_E

    def chunk_body(j, carry):
        eoff = pl.multiple_of(base + j * 512, 8)
        pltpu.sync_copy(src_hbm.at[pl.ds(eoff, 512)], s_v)
        pltpu.sync_copy(dst_hbm.at[pl.ds(eoff, 512)], d_v)

        def row_body(v, carry2):
            off = pl.multiple_of(v * 16, 8)
            vd = d_v[pl.ds(off, 16)]
            valid = (j * 512 + off + lane) < NV
            f = lax.shift_right_logical(vd, 8)
            occ, last = plsc.scan_count(f, valid)
            cnt = plsc.load_gather(cf_v, [f])
            plsc.store_scatter(cf_v, [f], cnt + occ, mask=last)
            docc, dlast = plsc.scan_count(vd, valid)
            dcnt = plsc.load_gather(deg_v, [vd])
            plsc.store_scatter(deg_v, [vd], dcnt + docc, mask=dlast)
            return carry2

        return lax.fori_loop(0, 32, row_body, carry)

    lax.fori_loop(0, 98, chunk_body, 0)

    pltpu.sync_copy(cf_v, counts_hbm.at[pl.ds(pl.multiple_of(w * NFP, 8), NFP)])
    pltpu.sync_copy(deg_v, degp_hbm.at[w])


_count_call = pl.kernel(
    _count_body,
    out_type=[
        jax.ShapeDtypeStruct((32 * NFP,), jnp.int32),
        jax.ShapeDtypeStruct((32, NDEGT), jnp.int32),
    ],
    mesh=_MESH,
    scratch_types=[
        pltpu.VMEM((512,), jnp.int32),
        pltpu.VMEM((512,), jnp.int32),
        pltpu.VMEM((NFP,), jnp.int32),
        pltpu.VMEM((NDEGT,), jnp.int32),
    ],
    compiler_params=_CP,
)


def _plan_body(counts_ref, bases_ref, meta_ref):
    cnts = counts_ref[...].astype(jnp.float32)
    totals = jnp.sum(cnts, axis=0, keepdims=True)
    alloc = jnp.floor((totals + 7.0) * 0.125) * 8.0
    ri = lax.broadcasted_iota(jnp.int32, (NFP, NFP), 0)
    ci = lax.broadcasted_iota(jnp.int32, (NFP, NFP), 1)
    m_excl = (ri < ci).astype(jnp.float32)
    segstart = jnp.dot(alloc, m_excl,
                       preferred_element_type=jnp.float32)
    ri32 = lax.broadcasted_iota(jnp.int32, (32, 32), 0)
    ci32 = lax.broadcasted_iota(jnp.int32, (32, 32), 1)
    a_excl = (ci32 < ri32).astype(jnp.float32)
    colcum = jnp.dot(a_excl, cnts,
                     preferred_element_type=jnp.float32)
    bases_ref[...] = (segstart + colcum).astype(jnp.int32)
    rio = lax.broadcasted_iota(jnp.int32, (8, NFP), 0)
    seg_b = jnp.broadcast_to(segstart, (8, NFP))
    tot_b = jnp.broadcast_to(totals, (8, NFP))
    meta_ref[...] = jnp.where(
        rio == 0, seg_b, jnp.where(rio == 1, tot_b, 0.0)).astype(jnp.int32)


def _plan(counts2d):
    return pl.pallas_call(
        _plan_body,
        out_shape=(
            jax.ShapeDtypeStruct((32, NFP), jnp.int32),
            jax.ShapeDtypeStruct((8, NFP), jnp.int32),
        ),
    )(counts2d)


def _place_body(src_hbm, dst_hbm, zcf_hbm, bases_hbm, lists_hbm,
                s_v, d_v, cf_v, b_v, pkst, post, sem):
    c = lax.axis_index("c")
    s = lax.axis_index("s")
    w = c * 16 + s
    lane = lax.iota(jnp.int32, 16)

    pltpu.sync_copy(zcf_hbm, cf_v)
    pltpu.sync_copy(bases_hbm.at[pl.ds(pl.multiple_of(w * NFP, 8), NFP)], b_v)
    base = w * TILE_E

    def chunk_body(j, carry):
        eoff = pl.multiple_of(base + j * 512, 8)
        pltpu.sync_copy(src_hbm.at[pl.ds(eoff, 512)], s_v)
        pltpu.sync_copy(dst_hbm.at[pl.ds(eoff, 512)], d_v)

        def row_body(v, carry2):
            off = pl.multiple_of(v * 16, 8)
            vd = d_v[pl.ds(off, 16)]
            vs = s_v[pl.ds(off, 16)]
            valid = (j * 512 + off + lane) < NV
            f = lax.shift_right_logical(vd, 8)
            occ, last = plsc.scan_count(f, valid)
            cnt = plsc.load_gather(cf_v, [f])
            plsc.store_scatter(cf_v, [f], cnt + occ, mask=last)
            gb = plsc.load_gather(b_v, [f])
            pos = jnp.where(valid, gb + cnt + occ - 1, -1)
            pk = jnp.bitwise_or(vs, jnp.left_shift(jnp.bitwise_and(vd, 255), 17))
            pkst[pl.ds(off, 16)] = pk
            post[pl.ds(off, 16)] = pos
            return carry2

        lax.fori_loop(0, 32, row_body, carry)
        pltpu.async_copy(
            pkst, lists_hbm.at[plsc.Indices(post, ignored_value=-1)], sem
        ).wait()
        return carry

    lax.fori_loop(0, 98, chunk_body, 0)


_place_call = pl.kernel(
    _place_body,
    out_type=jax.ShapeDtypeStruct((LLEN,), jnp.int32),
    mesh=_MESH,
    scratch_types=[
        pltpu.VMEM((512,), jnp.int32),
        pltpu.VMEM((512,), jnp.int32),
        pltpu.VMEM((NFP,), jnp.int32),
        pltpu.VMEM((NFP,), jnp.int32),
        pltpu.VMEM((512,), jnp.int32),
        pltpu.VMEM((512,), jnp.int32),
        pltpu.SemaphoreType.DMA,
    ],
    compiler_params=_CP,
)


def _make_scatter(D):
    def body(g_hbm, lists_hbm, meta_hbm, zacc_hbm, out_hbm,
             pk_v, gidx_v, mt_v, rows_v, acc_v, sem):
        c = lax.axis_index("c")
        s = lax.axis_index("s")
        w = c * 16 + s
        lane = lax.iota(jnp.int32, 16)
        rowidx = [lane + v8 * 16 for v8 in range(8)]

        pltpu.sync_copy(meta_hbm, mt_v)
        for r in range(13):
            fidx = r * 32 + w

            @pl.when(fidx <= NF - 1)
            def _chunk():
                pltpu.sync_copy(zacc_hbm, acc_v.at[pl.ds(0, 128)])
                pltpu.sync_copy(zacc_hbm, acc_v.at[pl.ds(128, 128)])
                fvec = jnp.zeros((16,), jnp.int32) + fidx
                sb_v = plsc.load_gather(mt_v, [fvec])
                ln_v = plsc.load_gather(mt_v, [fvec + NFP])
                sbase = jnp.sum(jnp.where(lane == 0, sb_v, 0))
                n = jnp.sum(jnp.where(lane == 0, ln_v, 0))
                nch = lax.div(n + 127, 128)

                def ch_body(j, carry):
                    joff = pl.multiple_of(sbase + j * 128, 8)
                    pltpu.sync_copy(lists_hbm.at[pl.ds(joff, 128)], pk_v)
                    rem = n - j * 128
                    dvs = []
                    for v8 in range(8):
                        pkv = pk_v[pl.ds(v8 * 16, 16)]
                        m = (lane + v8 * 16) < rem
                        sv = jnp.where(m, jnp.bitwise_and(pkv, 131071), 0)
                        dv = jnp.where(m, lax.shift_right_logical(pkv, 17), CH)
                        gidx_v[pl.ds(v8 * 16, 16)] = sv
                        dvs.append(dv)
                    pltpu.async_copy(g_hbm.at[gidx_v], rows_v, sem).wait()

                    def col_body(cc, carry2):
                        cful = jnp.zeros((16,), jnp.int32) + cc
                        for v8 in range(8):
                            vals = plsc.load_gather(rows_v, [rowidx[v8], cful])
                            plsc.addupdate_scatter(
                                acc_v, [carry2[v8], cful], vals)
                        return carry2

                    lax.fori_loop(0, D, col_body, tuple(dvs))
                    return carry

                lax.fori_loop(0, nch, ch_body, 0)
                ooff = pl.multiple_of(fidx * CH, 8)
                pltpu.sync_copy(acc_v.at[pl.ds(0, 128)],
                                out_hbm.at[pl.ds(ooff, 128)])
                pltpu.sync_copy(acc_v.at[pl.ds(128, 128)],
                                out_hbm.at[pl.ds(ooff + 128, 128)])

    return pl.kernel(
        body,
        out_type=jax.ShapeDtypeStruct((NACC, D), jnp.float32),
        mesh=_MESH,
        scratch_types=[
            pltpu.VMEM((128,), jnp.int32),
            pltpu.VMEM((128,), jnp.int32),
            pltpu.VMEM((2 * NFP,), jnp.int32),
            pltpu.VMEM((128, D), jnp.float32),
            pltpu.VMEM((CH + 8, D), jnp.float32),
            pltpu.SemaphoreType.DMA,
        ],
        compiler_params=_CP,
    )


_scatter_d2 = _make_scatter(D2)


def _deg_body(degp_ref, o_ref):
    total = jnp.sum(degp_ref[...], axis=0, keepdims=True)
    o_ref[...] = lax.rsqrt(total.astype(jnp.float32) + 1.0)


def _dinv(degp):
    bw = 2944
    return pl.pallas_call(
        _deg_body,
        grid=(NDEGT // bw,),
        in_specs=[pl.BlockSpec((32, bw), lambda i: (0, i))],
        out_specs=pl.BlockSpec((1, bw), lambda i: (0, i)),
        out_shape=jax.ShapeDtypeStruct((1, NDEGT), jnp.float32),
    )(degp)


def _mm_scale_body(x_ref, w_ref, dinv_ref, o_ref):
    o_ref[...] = (
        jnp.dot(x_ref[...], w_ref[...], preferred_element_type=jnp.float32)
        * dinv_ref[...]
    )


def _mm_scale(x, w, dinv):
    n, din = x.shape
    dout = w.shape[1]
    return pl.pallas_call(
        _mm_scale_body,
        grid=(n // BR,),
        in_specs=[
            pl.BlockSpec((BR, din), lambda i: (i, 0)),
            pl.BlockSpec((din, dout), lambda i: (0, 0)),
            pl.BlockSpec((BR, 1), lambda i: (i, 0)),
        ],
        out_specs=pl.BlockSpec((BR, dout), lambda i: (i, 0)),
        out_shape=jax.ShapeDtypeStruct((n, dout), jnp.float32),
    )(x, w, dinv)


def _mid_body(acc_ref, u_ref, dinv_ref, b1_ref, w1_ref, w2_ref, o_ref):
    agg = (acc_ref[...] + u_ref[...]) * dinv_ref[...]
    h = jnp.maximum(
        jnp.dot(agg, w1_ref[...], preferred_element_type=jnp.float32)
        + b1_ref[...], 0.0)
    o_ref[...] = (
        jnp.dot(h, w2_ref[...], preferred_element_type=jnp.float32)
        * dinv_ref[...]
    )


def _mid(acc, u, dinv, b1, w1, w2):
    n, din = u.shape
    dhid = w1.shape[1]
    dout = w2.shape[1]
    return pl.pallas_call(
        _mid_body,
        grid=(n // BR,),
        in_specs=[
            pl.BlockSpec((BR, din), lambda i: (i, 0)),
            pl.BlockSpec((BR, din), lambda i: (i, 0)),
            pl.BlockSpec((BR, 1), lambda i: (i, 0)),
            pl.BlockSpec((1, dhid), lambda i: (0, 0)),
            pl.BlockSpec((din, dhid), lambda i: (0, 0)),
            pl.BlockSpec((dhid, dout), lambda i: (0, 0)),
        ],
        out_specs=pl.BlockSpec((BR, dout), lambda i: (i, 0)),
        out_shape=jax.ShapeDtypeStruct((n, dout), jnp.float32),
    )(acc, u, dinv, b1.reshape(1, dhid), w1, w2)


def _final_body(acc_ref, g_ref, dinv_ref, b_ref, o_ref):
    i = pl.program_id(0)
    h = jnp.maximum((acc_ref[...] + g_ref[...]) * dinv_ref[...] + b_ref[...], 0.0)
    part = jnp.sum(h, axis=0, keepdims=True) * (1.0 / N)

    @pl.when(i == 0)
    def _init():
        o_ref[...] = jnp.zeros_like(o_ref)

    o_ref[...] += part


def _final_mean(acc, g, dinv, b):
    n, d = g.shape
    return pl.pallas_call(
        _final_body,
        grid=(n // BR,),
        in_specs=[
            pl.BlockSpec((BR, d), lambda i: (i, 0)),
            pl.BlockSpec((BR, d), lambda i: (i, 0)),
            pl.BlockSpec((BR, 1), lambda i: (i, 0)),
            pl.BlockSpec((1, d), lambda i: (0, 0)),
        ],
        out_specs=pl.BlockSpec((1, d), lambda i: (0, 0)),
        out_shape=jax.ShapeDtypeStruct((1, d), jnp.float32),
    )(acc, g, dinv, b.reshape(1, d))


@jax.jit
def kernel(x, edge_index, W1, b1, W2, b2):
    w1p = jnp.pad(W1, ((0, DU - W1.shape[0]), (0, D1 - W1.shape[1])))
    b1p = jnp.pad(b1, (0, D1 - b1.shape[0]))
    eyep = jnp.pad(jnp.eye(W1.shape[0], dtype=jnp.float32),
                   ((0, 0), (0, DU - W1.shape[0])))
    w2p = jnp.pad(W2, ((0, D1 - W2.shape[0]), (0, D2 - W2.shape[1])))
    b2p = jnp.pad(b2, (0, D2 - b2.shape[0]))
    srcp = jnp.pad(edge_index[0], (0, EPAD - E))
    dstp = jnp.pad(edge_index[1], (0, EPAD - E))
    zcf = jnp.zeros((NFP,), jnp.int32)
    zdeg = jnp.zeros((NDEGT,), jnp.int32)

    counts, degp = _count_call(srcp, dstp, zcf, zdeg)
    bases, meta = _plan(counts.reshape(32, NFP))
    lists = _place_call(srcp, dstp, zcf, bases.reshape(-1))

    dinv = _dinv(degp)[:, :N].reshape(N, 1)
    u = _mm_scale(x, eyep, dinv)
    accu = _scatter_d2(u, lists, meta.reshape(-1)[:2 * NFP],
                       jnp.zeros((128, DU), jnp.float32))
    g2 = _mid(accu[:N], u, dinv, b1p, w1p, w2p)
    acc2 = _scatter_d2(g2, lists, meta.reshape(-1)[:2 * NFP],
                       jnp.zeros((128, D2), jnp.float32))
    return _final_mean(acc2[:N], g2, dinv, b2p)[:, :100]


# speedup vs baseline: 1.4259x; 1.0008x over previous
import jax
import jax.numpy as jnp
from jax import lax
from jax.experimental import pallas as pl
from jax.experimental.pallas import tpu as pltpu
from jax.experimental.pallas import tpu_sc as plsc

N = 100000
E = 1600000
BR = 2000

TILE_E = 50176
EPAD = 32 * TILE_E
NV = 50000
CH = 256
NF = 391
NFP = 400
NDEGT = 100096
LLEN = E + 3200
NACC = NF * CH
D1 = 256
D2 = 128
DU = 128

_MESH = plsc.VectorSubcoreMesh(core_axis_name="c", subcore_axis_name="s")
_CP = pltpu.CompilerParams(needs_layout_passes=False)


def _count_body(src_hbm, dst_hbm, zcf_hbm, zdeg_hbm, counts_hbm, degp_hbm,
                s_v, d_v, cf_v, deg_v):
    c = lax.axis_index("c")
    s = lax.axis_index("s")
    w = c * 16 + s
    lane = lax.iota(jnp.int32, 16)

    pltpu.sync_copy(zcf_hbm, cf_v)
    pltpu.sync_copy(zdeg_hbm, deg_v)
    base = w * TILE_E

    def chunk_body(j, carry):
        eoff = pl.multiple_of(base + j * 512, 8)
        pltpu.sync_copy(src_hbm.at[pl.ds(eoff, 512)], s_v)
        pltpu.sync_copy(dst_hbm.at[pl.ds(eoff, 512)], d_v)

        def row_body(v, carry2):
            off = pl.multiple_of(v * 16, 8)
            vd = d_v[pl.ds(off, 16)]
            valid = (j * 512 + off + lane) < NV
            f = lax.shift_right_logical(vd, 8)
            occ, last = plsc.scan_count(f, valid)
            cnt = plsc.load_gather(cf_v, [f])
            plsc.store_scatter(cf_v, [f], cnt + occ, mask=last)
            docc, dlast = plsc.scan_count(vd, valid)
            dcnt = plsc.load_gather(deg_v, [vd])
            plsc.store_scatter(deg_v, [vd], dcnt + docc, mask=dlast)
            return carry2

        return lax.fori_loop(0, 32, row_body, carry)

    lax.fori_loop(0, 98, chunk_body, 0)

    pltpu.sync_copy(cf_v, counts_hbm.at[pl.ds(pl.multiple_of(w * NFP, 8), NFP)])
    pltpu.sync_copy(deg_v, degp_hbm.at[w])


_count_call = pl.kernel(
    _count_body,
    out_type=[
        jax.ShapeDtypeStruct((32 * NFP,), jnp.int32),
        jax.ShapeDtypeStruct((32, NDEGT), jnp.int32),
    ],
    mesh=_MESH,
    scratch_types=[
        pltpu.VMEM((512,), jnp.int32),
        pltpu.VMEM((512,), jnp.int32),
        pltpu.VMEM((NFP,), jnp.int32),
        pltpu.VMEM((NDEGT,), jnp.int32),
    ],
    compiler_params=_CP,
)


def _plan_body(counts_ref, bases_ref, meta_ref):
    cnts = counts_ref[...].astype(jnp.float32)
    totals = jnp.sum(cnts, axis=0, keepdims=True)
    alloc = jnp.floor((totals + 7.0) * 0.125) * 8.0
    ri = lax.broadcasted_iota(jnp.int32, (NFP, NFP), 0)
    ci = lax.broadcasted_iota(jnp.int32, (NFP, NFP), 1)
    m_excl = (ri < ci).astype(jnp.float32)
    segstart = jnp.dot(alloc, m_excl,
                       preferred_element_type=jnp.float32)
    ri32 = lax.broadcasted_iota(jnp.int32, (32, 32), 0)
    ci32 = lax.broadcasted_iota(jnp.int32, (32, 32), 1)
    a_excl = (ci32 < ri32).astype(jnp.float32)
    colcum = jnp.dot(a_excl, cnts,
                     preferred_element_type=jnp.float32)
    bases_ref[...] = (segstart + colcum).astype(jnp.int32)
    rio = lax.broadcasted_iota(jnp.int32, (8, NFP), 0)
    seg_b = jnp.broadcast_to(segstart, (8, NFP))
    tot_b = jnp.broadcast_to(totals, (8, NFP))
    meta_ref[...] = jnp.where(
        rio == 0, seg_b, jnp.where(rio == 1, tot_b, 0.0)).astype(jnp.int32)


def _plan(counts2d):
    return pl.pallas_call(
        _plan_body,
        out_shape=(
            jax.ShapeDtypeStruct((32, NFP), jnp.int32),
            jax.ShapeDtypeStruct((8, NFP), jnp.int32),
        ),
    )(counts2d)


def _place_body(src_hbm, dst_hbm, zcf_hbm, bases_hbm, lists_hbm,
                s_v, d_v, cf_v, b_v, pkst, post, sem):
    c = lax.axis_index("c")
    s = lax.axis_index("s")
    w = c * 16 + s
    lane = lax.iota(jnp.int32, 16)

    pltpu.sync_copy(zcf_hbm, cf_v)
    pltpu.sync_copy(bases_hbm.at[pl.ds(pl.multiple_of(w * NFP, 8), NFP)], b_v)
    base = w * TILE_E

    def chunk_body(j, carry):
        eoff = pl.multiple_of(base + j * 512, 8)
        pltpu.sync_copy(src_hbm.at[pl.ds(eoff, 512)], s_v)
        pltpu.sync_copy(dst_hbm.at[pl.ds(eoff, 512)], d_v)

        def row_body(v, carry2):
            off = pl.multiple_of(v * 16, 8)
            vd = d_v[pl.ds(off, 16)]
            vs = s_v[pl.ds(off, 16)]
            valid = (j * 512 + off + lane) < NV
            f = lax.shift_right_logical(vd, 8)
            occ, last = plsc.scan_count(f, valid)
            cnt = plsc.load_gather(cf_v, [f])
            plsc.store_scatter(cf_v, [f], cnt + occ, mask=last)
            gb = plsc.load_gather(b_v, [f])
            pos = jnp.where(valid, gb + cnt + occ - 1, -1)
            pk = jnp.bitwise_or(vs, jnp.left_shift(jnp.bitwise_and(vd, 255), 17))
            pkst[pl.ds(off, 16)] = pk
            post[pl.ds(off, 16)] = pos
            return carry2

        lax.fori_loop(0, 32, row_body, carry)
        pltpu.async_copy(
            pkst, lists_hbm.at[plsc.Indices(post, ignored_value=-1)], sem
        ).wait()
        return carry

    lax.fori_loop(0, 98, chunk_body, 0)


_place_call = pl.kernel(
    _place_body,
    out_type=jax.ShapeDtypeStruct((LLEN,), jnp.int32),
    mesh=_MESH,
    scratch_types=[
        pltpu.VMEM((512,), jnp.int32),
        pltpu.VMEM((512,), jnp.int32),
        pltpu.VMEM((NFP,), jnp.int32),
        pltpu.VMEM((NFP,), jnp.int32),
        pltpu.VMEM((512,), jnp.int32),
        pltpu.VMEM((512,), jnp.int32),
        pltpu.SemaphoreType.DMA,
    ],
    compiler_params=_CP,
)


def _make_scatter(D):
    def body(g_hbm, lists_hbm, meta_hbm, zacc_hbm, out_hbm,
             pk_v, gidx_v, mt_v, rows_v, acc_v, sem):
        c = lax.axis_index("c")
        s = lax.axis_index("s")
        w = s * 2 + c
        lane = lax.iota(jnp.int32, 16)
        rowidx = [lane + v8 * 16 for v8 in range(8)]

        pltpu.sync_copy(meta_hbm, mt_v)
        for r in range(13):
            fidx = r * 32 + w

            @pl.when(fidx <= NF - 1)
            def _chunk():
                pltpu.sync_copy(zacc_hbm, acc_v.at[pl.ds(0, 128)])
                pltpu.sync_copy(zacc_hbm, acc_v.at[pl.ds(128, 128)])
                fvec = jnp.zeros((16,), jnp.int32) + fidx
                sb_v = plsc.load_gather(mt_v, [fvec])
                ln_v = plsc.load_gather(mt_v, [fvec + NFP])
                sbase = jnp.sum(jnp.where(lane == 0, sb_v, 0))
                n = jnp.sum(jnp.where(lane == 0, ln_v, 0))
                nch = lax.div(n + 127, 128)

                def ch_body(j, carry):
                    joff = pl.multiple_of(sbase + j * 128, 8)
                    pltpu.sync_copy(lists_hbm.at[pl.ds(joff, 128)], pk_v)
                    rem = n - j * 128
                    dvs = []
                    for v8 in range(8):
                        pkv = pk_v[pl.ds(v8 * 16, 16)]
                        m = (lane + v8 * 16) < rem
                        sv = jnp.where(m, jnp.bitwise_and(pkv, 131071), 0)
                        dv = jnp.where(m, lax.shift_right_logical(pkv, 17), CH)
                        gidx_v[pl.ds(v8 * 16, 16)] = sv
                        dvs.append(dv)
                    pltpu.async_copy(g_hbm.at[gidx_v], rows_v, sem).wait()

                    def col_body(cc, carry2):
                        cful = jnp.zeros((16,), jnp.int32) + cc
                        for v8 in range(8):
                            vals = plsc.load_gather(rows_v, [rowidx[v8], cful])
                            plsc.addupdate_scatter(
                                acc_v, [carry2[v8], cful], vals)
                        return carry2

                    lax.fori_loop(0, D, col_body, tuple(dvs))
                    return carry

                lax.fori_loop(0, nch, ch_body, 0)
                ooff = pl.multiple_of(fidx * CH, 8)
                pltpu.sync_copy(acc_v.at[pl.ds(0, 128)],
                                out_hbm.at[pl.ds(ooff, 128)])
                pltpu.sync_copy(acc_v.at[pl.ds(128, 128)],
                                out_hbm.at[pl.ds(ooff + 128, 128)])

    return pl.kernel(
        body,
        out_type=jax.ShapeDtypeStruct((NACC, D), jnp.float32),
        mesh=_MESH,
        scratch_types=[
            pltpu.VMEM((128,), jnp.int32),
            pltpu.VMEM((128,), jnp.int32),
            pltpu.VMEM((2 * NFP,), jnp.int32),
            pltpu.VMEM((128, D), jnp.float32),
            pltpu.VMEM((CH + 8, D), jnp.float32),
            pltpu.SemaphoreType.DMA,
        ],
        compiler_params=_CP,
    )


_scatter_d2 = _make_scatter(D2)


def _deg_body(degp_ref, o_ref):
    total = jnp.sum(degp_ref[...], axis=0, keepdims=True)
    o_ref[...] = lax.rsqrt(total.astype(jnp.float32) + 1.0)


def _dinv(degp):
    bw = 2944
    return pl.pallas_call(
        _deg_body,
        grid=(NDEGT // bw,),
        in_specs=[pl.BlockSpec((32, bw), lambda i: (0, i))],
        out_specs=pl.BlockSpec((1, bw), lambda i: (0, i)),
        out_shape=jax.ShapeDtypeStruct((1, NDEGT), jnp.float32),
    )(degp)


def _mm_scale_body(x_ref, w_ref, dinv_ref, o_ref):
    o_ref[...] = (
        jnp.dot(x_ref[...], w_ref[...], preferred_element_type=jnp.float32)
        * dinv_ref[...]
    )


def _mm_scale(x, w, dinv):
    n, din = x.shape
    dout = w.shape[1]
    return pl.pallas_call(
        _mm_scale_body,
        grid=(n // BR,),
        in_specs=[
            pl.BlockSpec((BR, din), lambda i: (i, 0)),
            pl.BlockSpec((din, dout), lambda i: (0, 0)),
            pl.BlockSpec((BR, 1), lambda i: (i, 0)),
        ],
        out_specs=pl.BlockSpec((BR, dout), lambda i: (i, 0)),
        out_shape=jax.ShapeDtypeStruct((n, dout), jnp.float32),
    )(x, w, dinv)


def _mid_body(acc_ref, u_ref, dinv_ref, b1_ref, w1_ref, w2_ref, o_ref):
    agg = (acc_ref[...] + u_ref[...]) * dinv_ref[...]
    h = jnp.maximum(
        jnp.dot(agg, w1_ref[...], preferred_element_type=jnp.float32)
        + b1_ref[...], 0.0)
    o_ref[...] = (
        jnp.dot(h, w2_ref[...], preferred_element_type=jnp.float32)
        * dinv_ref[...]
    )


def _mid(acc, u, dinv, b1, w1, w2):
    n, din = u.shape
    dhid = w1.shape[1]
    dout = w2.shape[1]
    return pl.pallas_call(
        _mid_body,
        grid=(n // BR,),
        in_specs=[
            pl.BlockSpec((BR, din), lambda i: (i, 0)),
            pl.BlockSpec((BR, din), lambda i: (i, 0)),
            pl.BlockSpec((BR, 1), lambda i: (i, 0)),
            pl.BlockSpec((1, dhid), lambda i: (0, 0)),
            pl.BlockSpec((din, dhid), lambda i: (0, 0)),
            pl.BlockSpec((dhid, dout), lambda i: (0, 0)),
        ],
        out_specs=pl.BlockSpec((BR, dout), lambda i: (i, 0)),
        out_shape=jax.ShapeDtypeStruct((n, dout), jnp.float32),
    )(acc, u, dinv, b1.reshape(1, dhid), w1, w2)


def _final_body(acc_ref, g_ref, dinv_ref, b_ref, o_ref):
    i = pl.program_id(0)
    h = jnp.maximum((acc_ref[...] + g_ref[...]) * dinv_ref[...] + b_ref[...], 0.0)
    part = jnp.sum(h, axis=0, keepdims=True) * (1.0 / N)

    @pl.when(i == 0)
    def _init():
        o_ref[...] = jnp.zeros_like(o_ref)

    o_ref[...] += part


def _final_mean(acc, g, dinv, b):
    n, d = g.shape
    return pl.pallas_call(
        _final_body,
        grid=(n // BR,),
        in_specs=[
            pl.BlockSpec((BR, d), lambda i: (i, 0)),
            pl.BlockSpec((BR, d), lambda i: (i, 0)),
            pl.BlockSpec((BR, 1), lambda i: (i, 0)),
            pl.BlockSpec((1, d), lambda i: (0, 0)),
        ],
        out_specs=pl.BlockSpec((1, d), lambda i: (0, 0)),
        out_shape=jax.ShapeDtypeStruct((1, d), jnp.float32),
    )(acc, g, dinv, b.reshape(1, d))


@jax.jit
def kernel(x, edge_index, W1, b1, W2, b2):
    w1p = jnp.pad(W1, ((0, DU - W1.shape[0]), (0, D1 - W1.shape[1])))
    b1p = jnp.pad(b1, (0, D1 - b1.shape[0]))
    eyep = jnp.pad(jnp.eye(W1.shape[0], dtype=jnp.float32),
                   ((0, 0), (0, DU - W1.shape[0])))
    w2p = jnp.pad(W2, ((0, D1 - W2.shape[0]), (0, D2 - W2.shape[1])))
    b2p = jnp.pad(b2, (0, D2 - b2.shape[0]))
    srcp = jnp.pad(edge_index[0], (0, EPAD - E))
    dstp = jnp.pad(edge_index[1], (0, EPAD - E))
    zcf = jnp.zeros((NFP,), jnp.int32)
    zdeg = jnp.zeros((NDEGT,), jnp.int32)

    counts, degp = _count_call(srcp, dstp, zcf, zdeg)
    bases, meta = _plan(counts.reshape(32, NFP))
    lists = _place_call(srcp, dstp, zcf, bases.reshape(-1))

    dinv = _dinv(degp)[:, :N].reshape(N, 1)
    u = _mm_scale(x, eyep, dinv)
    accu = _scatter_d2(u, lists, meta.reshape(-1)[:2 * NFP],
                       jnp.zeros((128, DU), jnp.float32))
    g2 = _mid(accu[:N], u, dinv, b1p, w1p, w2p)
    acc2 = _scatter_d2(g2, lists, meta.reshape(-1)[:2 * NFP],
                       jnp.zeros((128, D2), jnp.float32))
    return _final_mean(acc2[:N], g2, dinv, b2p)[:, :100]
